# baseline (speedup 1.0000x reference)
import functools

import jax
import jax.numpy as jnp
from jax import lax
from jax.experimental import pallas as pl
from jax.experimental.pallas import tpu as pltpu
from jax.experimental.pallas import tpu_sc as plsc

_NUM_DENSE = 1024
_IFEAT = 64
_ICTX = 16
_HID = 128
_SPD = 32

_NC = 2
_NS = 16
_NW = _NC * _NS
_LANES = 16
_CH = 128


def _sc_gather_call(tokens_flat, ifeatures, sparse_table, dense_table, n):
    pw = n // _NW
    nch = pw // _CH

    mesh = plsc.VectorSubcoreMesh(core_axis_name="c", subcore_axis_name="s")

    @functools.partial(
        pl.kernel,
        mesh=mesh,
        compiler_params=pltpu.CompilerParams(use_tc_tiling_on_sc=False),
        out_type=[
            jax.ShapeDtypeStruct((n, _IFEAT), jnp.float32),
            jax.ShapeDtypeStruct((n, _SPD), jnp.float32),
            jax.ShapeDtypeStruct((n, _HID), jnp.float32),
            jax.ShapeDtypeStruct((n,), jnp.float32),
        ],
        scratch_types=[
            pltpu.VMEM((_CH,), jnp.int32),
            pltpu.VMEM((_CH,), jnp.int32),
            pltpu.VMEM((_CH,), jnp.int32),
            pltpu.VMEM((_CH,), jnp.float32),
            pltpu.VMEM((_CH, _IFEAT), jnp.float32),
            pltpu.VMEM((_CH, _SPD), jnp.float32),
            pltpu.VMEM((_CH, _HID), jnp.float32),
            pltpu.SemaphoreType.DMA,
            pltpu.SemaphoreType.DMA,
            pltpu.SemaphoreType.DMA,
        ],
    )
    def sc_gather(tok_hbm, if_hbm, sp_hbm, dn_hbm,
                  if_out, sp_out, dn_out, mk_out,
                  tok_v, spi_v, dni_v, msk_v, if_rows, sp_rows, dn_rows,
                  sem0, sem1, sem2):
        wid = lax.axis_index("s") * _NC + lax.axis_index("c")
        base = wid * pw

        def chunk(c, carry):
            off = base + c * _CH
            pltpu.sync_copy(tok_hbm.at[pl.ds(off, _CH)], tok_v)
            for i in range(_CH // _LANES):
                sl = pl.ds(i * _LANES, _LANES)
                t = tok_v[sl]
                sp = jnp.maximum(t - _NUM_DENSE, 0)
                spi_v[sl] = sp
                dni_v[sl] = jnp.where(t > _NUM_DENSE, 0, t)
                msk_v[sl] = jnp.where(sp > 0, 1.0, 0.0).astype(jnp.float32)
            d0 = pltpu.async_copy(if_hbm.at[tok_v], if_rows, sem0)
            d1 = pltpu.async_copy(sp_hbm.at[spi_v], sp_rows, sem1)
            d2 = pltpu.async_copy(dn_hbm.at[dni_v], dn_rows, sem2)
            d0.wait()
            d1.wait()
            d2.wait()
            pltpu.sync_copy(if_rows, if_out.at[pl.ds(off, _CH)])
            pltpu.sync_copy(sp_rows, sp_out.at[pl.ds(off, _CH)])
            pltpu.sync_copy(dn_rows, dn_out.at[pl.ds(off, _CH)])
            pltpu.sync_copy(msk_v, mk_out.at[pl.ds(off, _CH)])
            return carry

        lax.fori_loop(0, nch, chunk, 0)

    return sc_gather(tokens_flat, ifeatures, sparse_table, dense_table)


def _tc_body(if_ref, ic_ref, sp_ref, dn_ref, mk_ref,
             wca_ref, wcb_ref, wup_ref, w1_ref, be_ref, out_ref):
    dot = functools.partial(
        jnp.dot, preferred_element_type=jnp.float32,
        precision=jax.lax.Precision.HIGHEST)
    acc = dot(if_ref[...], wca_ref[...]) + dot(ic_ref[...], wcb_ref[...])
    spv = dot(sp_ref[...], wup_ref[...])
    dn = dn_ref[...]
    m = mk_ref[...]
    tv = dn + m * (spv - dn)
    v = acc + dot(tv, w1_ref[...]) + be_ref[...]
    s = jnp.sum(v * v, axis=1, keepdims=True)
    nrm = jnp.maximum(jnp.sqrt(s), 1e-12)
    out_ref[...] = v / nrm


def _tc_call(if_g, ic2, sp_g, dn_g, mk2, wca, wcb, w_up, w1, beff, n):
    t = 512
    g = n // t
    const = lambda shape: pl.BlockSpec(shape, lambda i: (0, 0))
    row = lambda d: pl.BlockSpec((t, d), lambda i: (i, 0))
    return pl.pallas_call(
        _tc_body,
        grid=(g,),
        in_specs=[
            row(_IFEAT), row(_ICTX), row(_SPD), row(_HID), row(1),
            const((_IFEAT, _HID)), const((_ICTX, _HID)),
            const((_SPD, _HID)), const((_HID, _HID)), const((1, _HID)),
        ],
        out_specs=row(_HID),
        out_shape=jax.ShapeDtypeStruct((n, _HID), jnp.float32),
    )(if_g, ic2, sp_g, dn_g, mk2, wca, wcb, w_up, w1, beff)


def kernel(tokens, icontexts, ifeatures, dense_table, sparse_table,
           W_up, W_ac, b_ac, W_item, b_item):
    b, l = tokens.shape
    n = b * l

    w1 = W_item[:_HID]
    w2 = W_item[_HID:]
    wc = W_ac @ w2
    beff = (b_item + b_ac @ w2).reshape(1, _HID)
    wca, wcb = wc[:_IFEAT], wc[_IFEAT:]

    tokens_flat = tokens.reshape(n).astype(jnp.int32)
    if_g, sp_g, dn_g, mk = _sc_gather_call(
        tokens_flat, ifeatures, sparse_table, dense_table, n)
    out = _tc_call(if_g, icontexts.reshape(n, _ICTX), sp_g, dn_g,
                   mk.reshape(n, 1), wca, wcb, W_up, w1, beff, n)
    return out.reshape(b, l, _HID)


# speedup vs baseline: 4.5978x; 4.5978x over previous
import functools

import jax
import jax.numpy as jnp
from jax import lax
from jax.experimental import pallas as pl
from jax.experimental.pallas import tpu as pltpu
from jax.experimental.pallas import tpu_sc as plsc

_NUM_DENSE = 1024
_IFEAT = 64
_ICTX = 16
_HID = 128
_SPD = 32
_SPARSE_ROWS_HINT = None

_NC = 2
_NS = 16
_NW = _NC * _NS
_LANES = 16
_CH = 128


def _sc_gather_call(tokens3, if_pad, sp_pad, dense_table, n, sparse_rows):
    pw = n // _NW
    nch = pw // _CH
    ng = nch // 2
    spread = sparse_rows - 1 - _NUM_DENSE
    dn_rows_tot = dense_table.shape[0]

    mesh = plsc.VectorSubcoreMesh(core_axis_name="c", subcore_axis_name="s")

    @functools.partial(
        pl.kernel,
        mesh=mesh,
        out_type=[
            jax.ShapeDtypeStruct((n, _HID), jnp.float32),
            jax.ShapeDtypeStruct((n, _HID), jnp.float32),
            jax.ShapeDtypeStruct((n, _HID), jnp.float32),
            jax.ShapeDtypeStruct((_NW, nch, _CH), jnp.float32),
        ],
        scratch_types=[
            pltpu.VMEM((nch, _CH), jnp.int32),
            pltpu.VMEM((nch, _CH), jnp.int32),
            pltpu.VMEM((nch, _CH), jnp.int32),
            pltpu.VMEM((nch, _CH), jnp.float32),
            pltpu.VMEM((_CH, _HID), jnp.float32),
            pltpu.VMEM((_CH, _HID), jnp.float32),
            pltpu.VMEM((_CH, _HID), jnp.float32),
            pltpu.VMEM((_CH, _HID), jnp.float32),
            pltpu.VMEM((_CH, _HID), jnp.float32),
            pltpu.VMEM((_CH, _HID), jnp.float32),
            pltpu.SemaphoreType.DMA,
            pltpu.SemaphoreType.DMA,
            pltpu.SemaphoreType.DMA,
            pltpu.SemaphoreType.DMA,
            pltpu.SemaphoreType.DMA,
        ],
    )
    def sc_gather(tok_hbm, if_hbm, sp_hbm, dn_hbm,
                  if_out, sp_out, dn_out, mk_out,
                  tokall, spiall, dniall, mskall,
                  ifr0, spr0, dnr0, ifr1, spr1, dnr1,
                  gsem0, gsem1, ssem0, ssem1, msem):
        wid = lax.axis_index("s") * _NC + lax.axis_index("c")
        base = wid * pw

        pltpu.sync_copy(tok_hbm.at[wid], tokall)

        def idx_chunk(c, carry):
            for i in range(_CH // _LANES):
                sl = pl.ds(i * _LANES, _LANES)
                t = tokall[c, sl]
                spiall[c, sl] = jnp.where(t > _NUM_DENSE, t - _NUM_DENSE,
                                          t + spread)
                dniall[c, sl] = lax.rem(t, dn_rows_tot)
                mskall[c, sl] = jnp.where(t > _NUM_DENSE, 1.0, 0.0
                                          ).astype(jnp.float32)
            return carry

        lax.fori_loop(0, nch, idx_chunk, 0)

        pltpu.async_copy(mskall, mk_out.at[wid], msem)

        def fire(c, ifr, spr, dnr, gsem):
            pltpu.async_copy(if_hbm.at[tokall.at[c]], ifr, gsem)
            pltpu.async_copy(sp_hbm.at[spiall.at[c]], spr, gsem)
            pltpu.async_copy(dn_hbm.at[dniall.at[c]], dnr, gsem)

        def wait_gathers(c, ifr, spr, dnr, gsem):
            pltpu.make_async_copy(if_hbm.at[tokall.at[c]], ifr, gsem).wait()
            pltpu.make_async_copy(sp_hbm.at[spiall.at[c]], spr, gsem).wait()
            pltpu.make_async_copy(dn_hbm.at[dniall.at[c]], dnr, gsem).wait()

        def fire_scatters(c, ifr, spr, dnr, ssem):
            off = base + c * _CH
            pltpu.async_copy(ifr, if_out.at[pl.ds(off, _CH)], ssem)
            pltpu.async_copy(spr, sp_out.at[pl.ds(off, _CH)], ssem)
            pltpu.async_copy(dnr, dn_out.at[pl.ds(off, _CH)], ssem)

        def wait_scatters(c, ifr, spr, dnr, ssem):
            off = base + c * _CH
            pltpu.make_async_copy(ifr, if_out.at[pl.ds(off, _CH)], ssem).wait()
            pltpu.make_async_copy(spr, sp_out.at[pl.ds(off, _CH)], ssem).wait()
            pltpu.make_async_copy(dnr, dn_out.at[pl.ds(off, _CH)], ssem).wait()

        fire(0, ifr0, spr0, dnr0, gsem0)
        fire(1, ifr1, spr1, dnr1, gsem1)

        def group(g, carry):
            c0 = 2 * g
            c1 = c0 + 1
            wait_gathers(c0, ifr0, spr0, dnr0, gsem0)
            fire_scatters(c0, ifr0, spr0, dnr0, ssem0)
            wait_gathers(c1, ifr1, spr1, dnr1, gsem1)
            fire_scatters(c1, ifr1, spr1, dnr1, ssem1)

            @pl.when(g + 1 < ng)
            def _():
                wait_scatters(c0, ifr0, spr0, dnr0, ssem0)
                fire(c0 + 2, ifr0, spr0, dnr0, gsem0)
                wait_scatters(c1, ifr1, spr1, dnr1, ssem1)
                fire(c1 + 2, ifr1, spr1, dnr1, gsem1)

            return carry

        lax.fori_loop(0, ng, group, 0)

        last0 = nch - 2
        last1 = nch - 1
        wait_scatters(last0, ifr0, spr0, dnr0, ssem0)
        wait_scatters(last1, ifr1, spr1, dnr1, ssem1)
        pltpu.make_async_copy(mskall, mk_out.at[wid], msem).wait()

    return sc_gather(tokens3, if_pad, sp_pad, dense_table)


def _tc_body(if_ref, ic_ref, sp_ref, dn_ref, mk_ref,
             wca_ref, wcb_ref, wup_ref, w1_ref, be_ref, out_ref):
    dot = functools.partial(
        jnp.dot, preferred_element_type=jnp.float32,
        precision=jax.lax.Precision.HIGHEST)
    acc = dot(if_ref[...], wca_ref[...]) + dot(ic_ref[...], wcb_ref[...])
    spv = dot(sp_ref[...], wup_ref[...])
    tv = jnp.where(mk_ref[...] != 0.0, spv, dn_ref[...])
    v = acc + dot(tv, w1_ref[...]) + be_ref[...]
    s = jnp.sum(v * v, axis=1, keepdims=True)
    nrm = jnp.maximum(jnp.sqrt(s), 1e-12)
    out_ref[...] = v / nrm


def _tc_call(if_g, ic2, sp_g, dn_g, mk2, wca, wcb, wup, w1, beff, n):
    t = 512
    g = n // t
    const = lambda shape: pl.BlockSpec(shape, lambda i: (0, 0))
    row = lambda d: pl.BlockSpec((t, d), lambda i: (i, 0))
    return pl.pallas_call(
        _tc_body,
        grid=(g,),
        in_specs=[
            row(_HID), row(_ICTX), row(_HID), row(_HID), row(1),
            const((_HID, _HID)), const((_ICTX, _HID)),
            const((_HID, _HID)), const((_HID, _HID)), const((1, _HID)),
        ],
        out_specs=row(_HID),
        out_shape=jax.ShapeDtypeStruct((n, _HID), jnp.float32),
    )(if_g, ic2, sp_g, dn_g, mk2, wca, wcb, wup, w1, beff)


def kernel(tokens, icontexts, ifeatures, dense_table, sparse_table,
           W_up, W_ac, b_ac, W_item, b_item):
    b, l = tokens.shape
    n = b * l
    pw = n // _NW
    nch = pw // _CH
    sparse_rows = sparse_table.shape[0]

    w1 = W_item[:_HID]
    w2 = W_item[_HID:]
    wc = W_ac @ w2
    beff = (b_item + b_ac @ w2).reshape(1, _HID)
    wca = jnp.pad(wc[:_IFEAT], ((0, _HID - _IFEAT), (0, 0)))
    wcb = wc[_IFEAT:]
    wup = jnp.pad(W_up, ((0, _HID - _SPD), (0, 0)))

    if_pad = jnp.pad(ifeatures, ((0, 0), (0, _HID - _IFEAT)))
    sp_pad = jnp.pad(sparse_table, ((0, 0), (0, _HID - _SPD)))

    tokens3 = tokens.reshape(_NW, nch, _CH).astype(jnp.int32)
    if_g, sp_g, dn_g, mk = _sc_gather_call(
        tokens3, if_pad, sp_pad, dense_table, n, sparse_rows)
    out = _tc_call(if_g, icontexts.reshape(n, _ICTX), sp_g, dn_g,
                   mk.reshape(n, 1), wca, wcb, wup, w1, beff, n)
    return out.reshape(b, l, _HID)
